# Initial kernel scaffold; baseline (speedup 1.0000x reference)
import functools

import jax
import jax.numpy as jnp
from jax import lax
from jax.experimental import pallas as pl
from jax.experimental.pallas import tpu as pltpu
from jax.experimental.pallas import tpu_sc as plsc

EPS = 1e-5
NC = 2
NS = 16
NW = NC * NS


def _zero_fill(ref, nrows, ncols):
  def body(i, _):
    for c in range(ncols // 16):
      ref[i, pl.ds(c * 16, 16)] = jnp.zeros((16,), jnp.float32)
    return 0
  lax.fori_loop(0, nrows, body, 0)


def _pick_zchunk(rpt):
  for z in (125, 128, 64, 25, 20, 16, 10, 8, 5, 4, 2, 1):
    if rpt % z == 0 and z <= 128:
      return z
  return 1


def _make_seg_kernel(n, e, d, nrel):
  ept = e // NW
  k = 80 if ept % 80 == 0 else ept
  assert ept % k == 0 and k <= 128 and k % 8 == 0, (e, ept, k)
  nchunk = ept // k
  rpt = n // NS
  assert n % NS == 0
  zr = _pick_zchunk(rpt)
  mesh = plsc.VectorSubcoreMesh(core_axis_name="c", subcore_axis_name="s")

  @functools.partial(
      pl.kernel,
      out_type=jax.ShapeDtypeStruct((nrel, NC, n, d), jnp.float32),
      mesh=mesh,
      scratch_types=[
          pltpu.VMEM((k,), jnp.int32),
          pltpu.VMEM((k,), jnp.int32),
          pltpu.VMEM((k, d), jnp.float32),
          pltpu.VMEM((zr, d), jnp.float32),
          pltpu.VMEM_SHARED((n, d), jnp.float32),
          pltpu.SemaphoreType.DMA,
      ],
  )
  def seg(g_hbm, src_hbm, dst_hbm, out_hbm, sidx, didx, rows, zv, acc, sem):
    c = lax.axis_index("c")
    s = lax.axis_index("s")
    _zero_fill(zv, zr, d)
    for r in range(nrel):
      for zc in range(rpt // zr):
        pltpu.sync_copy(zv, acc.at[pl.ds(s * rpt + zc * zr, zr)])
      plsc.subcore_barrier()

      def chunk(j, _):
        base = r * e + (c * NS + s) * ept + j * k
        pltpu.sync_copy(src_hbm.at[pl.ds(base, k)], sidx)
        pltpu.sync_copy(dst_hbm.at[pl.ds(base, k)], didx)
        pltpu.async_copy(g_hbm.at[sidx], rows, sem).wait()
        pltpu.sync_copy(rows, acc.at[didx], add=True)
        return 0

      lax.fori_loop(0, nchunk, chunk, 0)
      plsc.subcore_barrier()
      pltpu.sync_copy(acc.at[pl.ds(s * rpt, rpt)],
                      out_hbm.at[r, c, pl.ds(s * rpt, rpt)])
      plsc.subcore_barrier()

  return seg


def _make_deg_kernel(n, e, nrel):
  d = 16
  ept = e // NW
  k = 80 if ept % 80 == 0 else ept
  assert ept % k == 0 and k <= 128 and k % 8 == 0
  nchunk = ept // k
  rpt = n // NS
  zr = _pick_zchunk(rpt)
  mesh = plsc.VectorSubcoreMesh(core_axis_name="c", subcore_axis_name="s")

  @functools.partial(
      pl.kernel,
      out_type=jax.ShapeDtypeStruct((nrel, NC, n, d), jnp.float32),
      mesh=mesh,
      scratch_types=[
          pltpu.VMEM((k,), jnp.int32),
          pltpu.VMEM((k, d), jnp.float32),
          pltpu.VMEM((zr, d), jnp.float32),
          pltpu.VMEM_SHARED((n, d), jnp.float32),
      ],
  )
  def deg(dst_hbm, out_hbm, didx, ones_v, zv, acc):
    c = lax.axis_index("c")
    s = lax.axis_index("s")
    _zero_fill(zv, zr, d)

    def fill_ones(i, _):
      ones_v[i, pl.ds(0, 16)] = jnp.ones((16,), jnp.float32)
      return 0
    lax.fori_loop(0, k, fill_ones, 0)

    for r in range(nrel):
      for zc in range(rpt // zr):
        pltpu.sync_copy(zv, acc.at[pl.ds(s * rpt + zc * zr, zr)])
      plsc.subcore_barrier()

      def chunk(j, _):
        base = r * e + (c * NS + s) * ept + j * k
        pltpu.sync_copy(dst_hbm.at[pl.ds(base, k)], didx)
        pltpu.sync_copy(ones_v, acc.at[didx], add=True)
        return 0

      lax.fori_loop(0, nchunk, chunk, 0)
      plsc.subcore_barrier()
      pltpu.sync_copy(acc.at[pl.ds(s * rpt, rpt)],
                      out_hbm.at[r, c, pl.ds(s * rpt, rpt)])
      plsc.subcore_barrier()

  return deg


def _make_gather_kernel(nrows_tab, d, nidx):
  per_tile = nidx // NW
  assert nidx % NW == 0 and per_tile % 8 == 0
  k = 128 if per_tile % 128 == 0 else per_tile
  assert per_tile % k == 0 and k <= 128
  nchunk = per_tile // k
  mesh = plsc.VectorSubcoreMesh(core_axis_name="c", subcore_axis_name="s")

  @functools.partial(
      pl.kernel,
      out_type=jax.ShapeDtypeStruct((nidx, d), jnp.float32),
      mesh=mesh,
      scratch_types=[
          pltpu.VMEM((k,), jnp.int32),
          pltpu.VMEM((k, d), jnp.float32),
          pltpu.SemaphoreType.DMA,
      ],
  )
  def gat(tab_hbm, idx_hbm, out_hbm, idxv, rows, sem):
    c = lax.axis_index("c")
    s = lax.axis_index("s")
    wid = c * NS + s
    for q in range(nchunk):
      base = wid * per_tile + q * k
      pltpu.sync_copy(idx_hbm.at[pl.ds(base, k)], idxv)
      pltpu.async_copy(tab_hbm.at[idxv], rows, sem).wait()
      pltpu.sync_copy(rows, out_hbm.at[pl.ds(base, k)])

  return gat


def _tc_pre(x, w1s, degp):
  n, feat = x.shape
  nrel, _, hid = w1s.shape

  def body(x_ref, w_ref, degp_ref, h1_ref, g1_ref, dinv_ref):
    xv = x_ref[...]
    w = w_ref[0]
    deg = degp_ref[0, 0, :, 0:1] + degp_ref[0, 1, :, 0:1]
    dinv = lax.rsqrt(jnp.maximum(deg, 1.0))
    h = jnp.dot(xv, w, preferred_element_type=jnp.float32)
    h1_ref[0] = h
    g1_ref[0] = h * dinv
    dinv_ref[0] = dinv

  return pl.pallas_call(
      body,
      grid=(nrel,),
      in_specs=[
          pl.BlockSpec((n, feat), lambda r: (0, 0)),
          pl.BlockSpec((1, feat, hid), lambda r: (r, 0, 0)),
          pl.BlockSpec((1, NC, n, 16), lambda r: (r, 0, 0, 0)),
      ],
      out_specs=[
          pl.BlockSpec((1, n, hid), lambda r: (r, 0, 0)),
          pl.BlockSpec((1, n, hid), lambda r: (r, 0, 0)),
          pl.BlockSpec((1, n, 1), lambda r: (r, 0, 0)),
      ],
      out_shape=[
          jax.ShapeDtypeStruct((nrel, n, hid), jnp.float32),
          jax.ShapeDtypeStruct((nrel, n, hid), jnp.float32),
          jax.ShapeDtypeStruct((nrel, n, 1), jnp.float32),
      ],
  )(x, w1s, degp)


def _tc_mid(p1, h1, dinv, k21, b1s, gm1s, bt1s, w2s):
  nrel, _, n, hid = p1.shape
  out = w2s.shape[2]

  def body(p_ref, h1_ref, dinv_ref, k2_ref, b_ref, gm_ref, bt_ref, w2_ref,
           h2_ref, g2_ref, loss_ref):
    p = p_ref[0]
    h = h1_ref[0]
    dinv = dinv_ref[0]
    k2 = k2_ref[0, 0]
    agg = (p[0] + p[1]) * dinv
    out1 = agg + k2 * h + b_ref[0][None, :]
    lap = (1.0 + k2) * h - agg
    loss_ref[0, 0] = jnp.mean(lap * lap)
    mu = jnp.mean(out1, axis=0, keepdims=True)
    var = jnp.mean((out1 - mu) ** 2, axis=0, keepdims=True)
    x1 = jnp.tanh(gm_ref[0][None, :] * (out1 - mu)
                  / jnp.sqrt(var + EPS) + bt_ref[0][None, :])
    h2 = jnp.dot(x1, w2_ref[0], preferred_element_type=jnp.float32)
    h2_ref[0] = h2
    g2_ref[0] = h2 * dinv

  return pl.pallas_call(
      body,
      grid=(nrel,),
      in_specs=[
          pl.BlockSpec((1, NC, n, hid), lambda r: (r, 0, 0, 0)),
          pl.BlockSpec((1, n, hid), lambda r: (r, 0, 0)),
          pl.BlockSpec((1, n, 1), lambda r: (r, 0, 0)),
          pl.BlockSpec((1, 1), lambda r: (r, 0)),
          pl.BlockSpec((1, hid), lambda r: (r, 0)),
          pl.BlockSpec((1, hid), lambda r: (r, 0)),
          pl.BlockSpec((1, hid), lambda r: (r, 0)),
          pl.BlockSpec((1, hid, out), lambda r: (r, 0, 0)),
      ],
      out_specs=[
          pl.BlockSpec((1, n, out), lambda r: (r, 0, 0)),
          pl.BlockSpec((1, n, out), lambda r: (r, 0, 0)),
          pl.BlockSpec((1, 1), lambda r: (r, 0)),
      ],
      out_shape=[
          jax.ShapeDtypeStruct((nrel, n, out), jnp.float32),
          jax.ShapeDtypeStruct((nrel, n, out), jnp.float32),
          jax.ShapeDtypeStruct((nrel, 1), jnp.float32),
      ],
  )(p1, h1, dinv, k21, b1s, gm1s, bt1s, w2s)


def _tc_post(p2, h2, dinv, k22, b2s, gm2s, bt2s):
  nrel, _, n, out = p2.shape

  def body(p_ref, h2_ref, dinv_ref, k2_ref, b_ref, gm_ref, bt_ref, x2_ref):
    p = p_ref[0]
    h = h2_ref[0]
    dinv = dinv_ref[0]
    k2 = k2_ref[0, 0]
    agg = (p[0] + p[1]) * dinv
    out2 = agg + k2 * h + b_ref[0][None, :]
    mu = jnp.mean(out2, axis=0, keepdims=True)
    var = jnp.mean((out2 - mu) ** 2, axis=0, keepdims=True)
    x2_ref[0] = jnp.tanh(gm_ref[0][None, :] * (out2 - mu)
                         / jnp.sqrt(var + EPS) + bt_ref[0][None, :])

  return pl.pallas_call(
      body,
      grid=(nrel,),
      in_specs=[
          pl.BlockSpec((1, NC, n, out), lambda r: (r, 0, 0, 0)),
          pl.BlockSpec((1, n, out), lambda r: (r, 0, 0)),
          pl.BlockSpec((1, n, 1), lambda r: (r, 0, 0)),
          pl.BlockSpec((1, 1), lambda r: (r, 0)),
          pl.BlockSpec((1, out), lambda r: (r, 0)),
          pl.BlockSpec((1, out), lambda r: (r, 0)),
          pl.BlockSpec((1, out), lambda r: (r, 0)),
      ],
      out_specs=[pl.BlockSpec((1, n, out), lambda r: (r, 0, 0))],
      out_shape=[jax.ShapeDtypeStruct((nrel, n, out), jnp.float32)],
  )(p2, h2, dinv, k22, b2s, gm2s, bt2s)[0]


def _tc_final(xb, losses):
  nrel, b, out = xb.shape

  def body(x_ref, loss_ref, emb_ref, lm_ref):
    r = pl.program_id(0)
    x = x_ref[0]
    m = jnp.max(x, axis=1, keepdims=True)
    lse = jnp.log(jnp.sum(jnp.exp(x - m), axis=1, keepdims=True))
    emb_ref[...] = x - m - lse

    @pl.when(r == 0)
    def _():
      lm_ref[0, 0] = 0.0
    lm_ref[0, 0] += loss_ref[0, 0] / nrel

  return pl.pallas_call(
      body,
      grid=(nrel,),
      in_specs=[
          pl.BlockSpec((1, b, out), lambda r: (r, 0, 0)),
          pl.BlockSpec((1, 1), lambda r: (r, 0)),
      ],
      out_specs=[
          pl.BlockSpec((b, out), lambda r: (0, r)),
          pl.BlockSpec((1, 1), lambda r: (0, 0)),
      ],
      out_shape=[
          jax.ShapeDtypeStruct((b, nrel * out), jnp.float32),
          jax.ShapeDtypeStruct((1, 1), jnp.float32),
      ],
  )(xb, losses)


def kernel(features, edge_index0, edge_index1, edge_index2, batch_nodes,
           params):
  n, _ = features.shape
  e = edge_index0.shape[1]
  b = batch_nodes.shape[0]
  nrel = 3
  hid = params[0]['W1'].shape[1]
  out = params[0]['W2'].shape[1]

  i32 = jnp.int32
  src_all = jnp.concatenate([
      edge_index0[0].astype(i32),
      edge_index1[0].astype(i32) + n,
      edge_index2[0].astype(i32) + 2 * n,
  ])
  dst_all = jnp.concatenate([
      edge_index0[1].astype(i32),
      edge_index1[1].astype(i32),
      edge_index2[1].astype(i32),
  ])
  bidx = jnp.concatenate(
      [batch_nodes.astype(i32) + r * n for r in range(nrel)])

  w1s = jnp.stack([p['W1'] for p in params])
  b1s = jnp.stack([p['b1'] for p in params])
  k21 = jnp.stack([p['k2_1'] for p in params]).reshape(nrel, 1)
  gm1s = jnp.stack([p['g1'] for p in params])
  bt1s = jnp.stack([p['be1'] for p in params])
  w2s = jnp.stack([p['W2'] for p in params])
  b2s = jnp.stack([p['b2'] for p in params])
  k22 = jnp.stack([p['k2_2'] for p in params]).reshape(nrel, 1)
  gm2s = jnp.stack([p['g2'] for p in params])
  bt2s = jnp.stack([p['be2'] for p in params])

  degp = _make_deg_kernel(n, e, nrel)(dst_all)
  h1, g1, dinv = _tc_pre(features, w1s, degp)
  p1 = _make_seg_kernel(n, e, hid, nrel)(
      g1.reshape(nrel * n, hid), src_all, dst_all)
  h2, g2, losses = _tc_mid(p1, h1, dinv, k21, b1s, gm1s, bt1s, w2s)
  p2 = _make_seg_kernel(n, e, out, nrel)(
      g2.reshape(nrel * n, out), src_all, dst_all)
  x2 = _tc_post(p2, h2, dinv, k22, b2s, gm2s, bt2s)
  xb = _make_gather_kernel(nrel * n, out, nrel * b)(
      x2.reshape(nrel * n, out), bidx)
  final, lossm = _tc_final(xb.reshape(nrel, b, out), losses)
  return final, lossm[0, 0]


# speedup vs baseline: 9.1780x; 9.1780x over previous
import functools

import jax
import jax.numpy as jnp
from jax import lax
from jax.experimental import pallas as pl
from jax.experimental.pallas import tpu as pltpu
from jax.experimental.pallas import tpu_sc as plsc

EPS = 1e-5
NC = 2
NS = 16
NW = NC * NS


def _zero_fill(ref, nrows, ncols):
  def body(i, _):
    for c in range(ncols // 16):
      ref[i, pl.ds(c * 16, 16)] = jnp.zeros((16,), jnp.float32)
    return 0
  lax.fori_loop(0, nrows, body, 0)


def _row_partition(n):
  fl = (n // NS) & ~7
  extra = n - NS * fl
  zr = 8
  for z in range(128, 7, -8):
    if fl % z == 0:
      zr = z
      break
  assert extra % 8 == 0 and fl % zr == 0
  return fl, extra, zr


def _make_seg_kernel(n, e, d, nrel):
  ept = e // NW
  k = 80 if ept % 80 == 0 else ept
  assert ept % k == 0 and k <= 128 and k % 8 == 0, (e, ept, k)
  nchunk = ept // k
  fl, extra, zr = _row_partition(n)
  mesh = plsc.VectorSubcoreMesh(core_axis_name="c", subcore_axis_name="s", num_cores=NC, num_subcores=NS)

  @functools.partial(
      pl.kernel,
      out_type=jax.ShapeDtypeStruct((nrel, NC, n, d), jnp.float32),
      mesh=mesh,
      scratch_types=[
          pltpu.VMEM((k,), jnp.int32),
          pltpu.VMEM((k,), jnp.int32),
          pltpu.VMEM((k, d), jnp.float32),
          pltpu.VMEM((zr, d), jnp.float32),
          pltpu.VMEM_SHARED((n, d), jnp.float32),
          pltpu.SemaphoreType.DMA,
      ],
  )
  def seg(g_hbm, src_hbm, dst_hbm, out_hbm, sidx, didx, rows, zv, acc, sem):
    c = lax.axis_index("c")
    s = lax.axis_index("s")
    _zero_fill(zv, zr, d)
    for r in range(nrel):
      for zc in range(fl // zr):
        pltpu.sync_copy(zv, acc.at[pl.ds(s * fl + zc * zr, zr)])
      if extra:
        @pl.when(s == NS - 1)
        def _():
          for zc in range(extra // 8):
            pltpu.sync_copy(zv.at[pl.ds(0, 8)],
                            acc.at[pl.ds(NS * fl + zc * 8, 8)])
      plsc.subcore_barrier()

      def chunk(j, _):
        base = r * e + (c * NS + s) * ept + j * k
        pltpu.sync_copy(src_hbm.at[pl.ds(base, k)], sidx)
        pltpu.sync_copy(dst_hbm.at[pl.ds(base, k)], didx)
        pltpu.async_copy(g_hbm.at[sidx], rows, sem).wait()
        pltpu.sync_copy(rows, acc.at[didx], add=True)
        return 0

      lax.fori_loop(0, nchunk, chunk, 0)
      plsc.subcore_barrier()
      pltpu.sync_copy(acc.at[pl.ds(s * fl, fl)],
                      out_hbm.at[r, c, pl.ds(s * fl, fl)])
      if extra:
        @pl.when(s == NS - 1)
        def _():
          pltpu.sync_copy(acc.at[pl.ds(NS * fl, extra)],
                          out_hbm.at[r, c, pl.ds(NS * fl, extra)])
      plsc.subcore_barrier()

  return seg


def _make_deg_kernel(n, e, nrel):
  d = 16
  ept = e // NW
  k = 80 if ept % 80 == 0 else ept
  assert ept % k == 0 and k <= 128 and k % 8 == 0
  nchunk = ept // k
  fl, extra, zr = _row_partition(n)
  seg_sz = n * d
  mesh = plsc.VectorSubcoreMesh(core_axis_name="c", subcore_axis_name="s", num_cores=NC, num_subcores=NS)

  @functools.partial(
      pl.kernel,
      out_type=jax.ShapeDtypeStruct((nrel * NC * seg_sz,), jnp.float32),
      mesh=mesh,
      scratch_types=[
          pltpu.VMEM((k,), jnp.int32),
          pltpu.VMEM((k,), jnp.float32),
          pltpu.VMEM((zr * d,), jnp.float32),
          pltpu.VMEM_SHARED((seg_sz,), jnp.float32),
      ],
  )
  def deg(dst16_hbm, out_hbm, didx, ones_v, zv, acc):
    c = lax.axis_index("c")
    s = lax.axis_index("s")

    def fill(i, _):
      zv[pl.ds(i * 16, 16)] = jnp.zeros((16,), jnp.float32)
      return 0
    lax.fori_loop(0, zr * d // 16, fill, 0)

    def fill_ones(i, _):
      ones_v[pl.ds(i * 16, 16)] = jnp.ones((16,), jnp.float32)
      return 0
    lax.fori_loop(0, k // 16, fill_ones, 0)

    for r in range(nrel):
      for zc in range(fl // zr):
        pltpu.sync_copy(zv, acc.at[pl.ds((s * fl + zc * zr) * d, zr * d)])
      if extra:
        @pl.when(s == NS - 1)
        def _():
          for zc in range(extra * d // (8 * d)):
            pltpu.sync_copy(zv.at[pl.ds(0, 8 * d)],
                            acc.at[pl.ds((NS * fl + zc * 8) * d, 8 * d)])
      plsc.subcore_barrier()

      def chunk(j, _):
        base = r * e + (c * NS + s) * ept + j * k
        pltpu.sync_copy(dst16_hbm.at[pl.ds(base, k)], didx)
        pltpu.sync_copy(ones_v, acc.at[didx], add=True)
        return 0

      lax.fori_loop(0, nchunk, chunk, 0)
      plsc.subcore_barrier()
      pltpu.sync_copy(
          acc.at[pl.ds(s * fl * d, fl * d)],
          out_hbm.at[pl.ds((r * NC + c) * seg_sz + s * fl * d, fl * d)])
      if extra:
        @pl.when(s == NS - 1)
        def _():
          pltpu.sync_copy(
              acc.at[pl.ds(NS * fl * d, extra * d)],
              out_hbm.at[pl.ds((r * NC + c) * seg_sz + NS * fl * d,
                               extra * d)])
      plsc.subcore_barrier()

  return deg


def _make_gather_kernel(nrows_tab, d, nidx):
  per_tile = nidx // NW
  assert nidx % NW == 0 and per_tile % 8 == 0
  k = 128 if per_tile % 128 == 0 else per_tile
  assert per_tile % k == 0 and k <= 128
  nchunk = per_tile // k
  mesh = plsc.VectorSubcoreMesh(core_axis_name="c", subcore_axis_name="s", num_cores=NC, num_subcores=NS)

  @functools.partial(
      pl.kernel,
      out_type=jax.ShapeDtypeStruct((nidx, d), jnp.float32),
      mesh=mesh,
      scratch_types=[
          pltpu.VMEM((k,), jnp.int32),
          pltpu.VMEM((k, d), jnp.float32),
          pltpu.SemaphoreType.DMA,
      ],
  )
  def gat(tab_hbm, idx_hbm, out_hbm, idxv, rows, sem):
    c = lax.axis_index("c")
    s = lax.axis_index("s")
    wid = c * NS + s
    for q in range(nchunk):
      base = wid * per_tile + q * k
      pltpu.sync_copy(idx_hbm.at[pl.ds(base, k)], idxv)
      pltpu.async_copy(tab_hbm.at[idxv], rows, sem).wait()
      pltpu.sync_copy(rows, out_hbm.at[pl.ds(base, k)])

  return gat


def _tc_comb(degp):
  nrel, _, n, dw = degp.shape

  def body(degp_ref, dinv_ref):
    deg = degp_ref[0, 0, :, 0:1] + degp_ref[0, 1, :, 0:1]
    dinv_ref[0] = lax.rsqrt(jnp.maximum(deg, 1.0))

  return pl.pallas_call(
      body,
      grid=(nrel,),
      in_specs=[pl.BlockSpec((1, NC, n, dw), lambda r: (r, 0, 0, 0))],
      out_specs=[pl.BlockSpec((1, n, 1), lambda r: (r, 0, 0))],
      out_shape=[jax.ShapeDtypeStruct((nrel, n, 1), jnp.float32)],
  )(degp)[0]


def _tc_pre(x, w1s, dinv):
  n, feat = x.shape
  nrel, _, hid = w1s.shape

  def body(x_ref, w_ref, dinv_ref, h1_ref, g1_ref):
    h = jnp.dot(x_ref[...], w_ref[0], preferred_element_type=jnp.float32)
    h1_ref[0] = h
    g1_ref[0] = h * dinv_ref[0]

  return pl.pallas_call(
      body,
      grid=(nrel,),
      in_specs=[
          pl.BlockSpec((n, feat), lambda r: (0, 0)),
          pl.BlockSpec((1, feat, hid), lambda r: (r, 0, 0)),
          pl.BlockSpec((1, n, 1), lambda r: (r, 0, 0)),
      ],
      out_specs=[
          pl.BlockSpec((1, n, hid), lambda r: (r, 0, 0)),
          pl.BlockSpec((1, n, hid), lambda r: (r, 0, 0)),
      ],
      out_shape=[
          jax.ShapeDtypeStruct((nrel, n, hid), jnp.float32),
          jax.ShapeDtypeStruct((nrel, n, hid), jnp.float32),
      ],
  )(x, w1s, dinv)


def _tc_scale(p, dinv):
  nrel, _, n, d = p.shape

  def body(p_ref, dinv_ref, a_ref):
    p0 = p_ref[0]
    a_ref[0] = (p0[0] + p0[1]) * dinv_ref[0]

  return pl.pallas_call(
      body,
      grid=(nrel,),
      in_specs=[
          pl.BlockSpec((1, NC, n, d), lambda r: (r, 0, 0, 0)),
          pl.BlockSpec((1, n, 1), lambda r: (r, 0, 0)),
      ],
      out_specs=[pl.BlockSpec((1, n, d), lambda r: (r, 0, 0))],
      out_shape=[jax.ShapeDtypeStruct((nrel, n, d), jnp.float32)],
  )(p, dinv)[0]


def _tc_mid(agg1, h1, dinv, k21, b1s, gm1s, bt1s):
  nrel, n, hid = agg1.shape

  def body(agg_ref, h1_ref, dinv_ref, k2_ref, b_ref, gm_ref, bt_ref,
           x1_ref, gx1_ref, loss_ref):
    r = pl.program_id(0)
    agg = agg_ref[0]
    h = h1_ref[0]
    k2 = k2_ref[r, 0]
    out1 = agg + k2 * h + b_ref[0, 0][None, :]
    lap = (1.0 + k2) * h - agg
    loss_ref[r, 0] = jnp.mean(lap * lap)
    mu = jnp.mean(out1, axis=0, keepdims=True)
    var = jnp.mean((out1 - mu) ** 2, axis=0, keepdims=True)
    x1 = jnp.tanh(gm_ref[0, 0][None, :] * (out1 - mu)
                  / jnp.sqrt(var + EPS) + bt_ref[0, 0][None, :])
    x1_ref[0] = x1
    gx1_ref[0] = x1 * dinv_ref[0]

  return pl.pallas_call(
      body,
      grid=(nrel,),
      in_specs=[
          pl.BlockSpec((1, n, hid), lambda r: (r, 0, 0)),
          pl.BlockSpec((1, n, hid), lambda r: (r, 0, 0)),
          pl.BlockSpec((1, n, 1), lambda r: (r, 0, 0)),
          pl.BlockSpec(memory_space=pltpu.SMEM),
          pl.BlockSpec((1, 1, hid), lambda r: (r, 0, 0)),
          pl.BlockSpec((1, 1, hid), lambda r: (r, 0, 0)),
          pl.BlockSpec((1, 1, hid), lambda r: (r, 0, 0)),
      ],
      out_specs=[
          pl.BlockSpec((1, n, hid), lambda r: (r, 0, 0)),
          pl.BlockSpec((1, n, hid), lambda r: (r, 0, 0)),
          pl.BlockSpec(memory_space=pltpu.SMEM),
      ],
      out_shape=[
          jax.ShapeDtypeStruct((nrel, n, hid), jnp.float32),
          jax.ShapeDtypeStruct((nrel, n, hid), jnp.float32),
          jax.ShapeDtypeStruct((nrel, 1), jnp.float32),
      ],
  )(agg1, h1, dinv, k21, b1s, gm1s, bt1s)


def _tc_post(q, x1, k22, b2s, w2s):
  nrel, n, hid = q.shape
  out = w2s.shape[2]

  def body(q_ref, x1_ref, k2_ref, b_ref, w2_ref, m_ref, mu_ref, var_ref):
    r = pl.program_id(0)
    k2 = k2_ref[r, 0]
    w2 = w2_ref[0]
    agg2 = jnp.dot(q_ref[0], w2, preferred_element_type=jnp.float32)
    h2 = jnp.dot(x1_ref[0], w2, preferred_element_type=jnp.float32)
    out2 = agg2 + k2 * h2 + b_ref[0, 0][None, :]
    mu = jnp.mean(out2, axis=0, keepdims=True)
    var = jnp.mean((out2 - mu) ** 2, axis=0, keepdims=True)
    mu_ref[0] = mu
    var_ref[0] = var
    m_ref[0] = jnp.concatenate([agg2, h2], axis=1)

  return pl.pallas_call(
      body,
      grid=(nrel,),
      in_specs=[
          pl.BlockSpec((1, n, hid), lambda r: (r, 0, 0)),
          pl.BlockSpec((1, n, hid), lambda r: (r, 0, 0)),
          pl.BlockSpec(memory_space=pltpu.SMEM),
          pl.BlockSpec((1, 1, out), lambda r: (r, 0, 0)),
          pl.BlockSpec((1, hid, out), lambda r: (r, 0, 0)),
      ],
      out_specs=[
          pl.BlockSpec((1, n, 2 * out), lambda r: (r, 0, 0)),
          pl.BlockSpec((1, 1, out), lambda r: (r, 0, 0)),
          pl.BlockSpec((1, 1, out), lambda r: (r, 0, 0)),
      ],
      out_shape=[
          jax.ShapeDtypeStruct((nrel, n, 2 * out), jnp.float32),
          jax.ShapeDtypeStruct((nrel, 1, out), jnp.float32),
          jax.ShapeDtypeStruct((nrel, 1, out), jnp.float32),
      ],
  )(q, x1, k22, b2s, w2s)


def _tc_final(mb, mu2, var2, k22, b2s, gm2s, bt2s, losses):
  nrel, b, two_out = mb.shape
  out = two_out // 2

  def body(mb_ref, mu_ref, var_ref, k2_ref, b_ref, gm_ref, bt_ref, loss_ref,
           emb_ref, lm_ref):
    r = pl.program_id(0)
    m = mb_ref[0]
    agg2 = m[:, :out]
    h2 = m[:, out:]
    k2 = k2_ref[r, 0]
    out2 = agg2 + k2 * h2 + b_ref[0, 0][None, :]
    mu = mu_ref[0]
    var = var_ref[0]
    x2 = jnp.tanh(gm_ref[0, 0][None, :] * (out2 - mu)
                  / jnp.sqrt(var + EPS) + bt_ref[0, 0][None, :])
    mx = jnp.max(x2, axis=1, keepdims=True)
    lse = jnp.log(jnp.sum(jnp.exp(x2 - mx), axis=1, keepdims=True))
    emb_ref[0] = x2 - mx - lse

    @pl.when(r == 0)
    def _():
      lm_ref[0, 0] = 0.0
    lm_ref[0, 0] += loss_ref[r, 0] / nrel

  return pl.pallas_call(
      body,
      grid=(nrel,),
      in_specs=[
          pl.BlockSpec((1, b, two_out), lambda r: (r, 0, 0)),
          pl.BlockSpec((1, 1, out), lambda r: (r, 0, 0)),
          pl.BlockSpec((1, 1, out), lambda r: (r, 0, 0)),
          pl.BlockSpec(memory_space=pltpu.SMEM),
          pl.BlockSpec((1, 1, out), lambda r: (r, 0, 0)),
          pl.BlockSpec((1, 1, out), lambda r: (r, 0, 0)),
          pl.BlockSpec((1, 1, out), lambda r: (r, 0, 0)),
          pl.BlockSpec(memory_space=pltpu.SMEM),
      ],
      out_specs=[
          pl.BlockSpec((1, b, out), lambda r: (r, 0, 0)),
          pl.BlockSpec(memory_space=pltpu.SMEM),
      ],
      out_shape=[
          jax.ShapeDtypeStruct((nrel, b, out), jnp.float32),
          jax.ShapeDtypeStruct((1, 1), jnp.float32),
      ],
  )(mb, mu2, var2, k22, b2s, gm2s, bt2s, losses)


def kernel(features, edge_index0, edge_index1, edge_index2, batch_nodes,
           params):
  n, _ = features.shape
  e = edge_index0.shape[1]
  b = batch_nodes.shape[0]
  nrel = 3
  hid = params[0]['W1'].shape[1]
  out = params[0]['W2'].shape[1]

  i32 = jnp.int32
  src_all = jnp.concatenate([
      edge_index0[0].astype(i32),
      edge_index1[0].astype(i32) + n,
      edge_index2[0].astype(i32) + 2 * n,
  ])
  dst_all = jnp.concatenate([
      edge_index0[1].astype(i32),
      edge_index1[1].astype(i32),
      edge_index2[1].astype(i32),
  ])
  bidx = jnp.concatenate(
      [batch_nodes.astype(i32) + r * n for r in range(nrel)])

  w1s = jnp.stack([p['W1'] for p in params])
  b1s = jnp.stack([p['b1'] for p in params]).reshape(nrel, 1, hid)
  k21 = jnp.stack([p['k2_1'] for p in params]).reshape(nrel, 1)
  gm1s = jnp.stack([p['g1'] for p in params]).reshape(nrel, 1, hid)
  bt1s = jnp.stack([p['be1'] for p in params]).reshape(nrel, 1, hid)
  w2s = jnp.stack([p['W2'] for p in params])
  b2s = jnp.stack([p['b2'] for p in params]).reshape(nrel, 1, out)
  k22 = jnp.stack([p['k2_2'] for p in params]).reshape(nrel, 1)
  gm2s = jnp.stack([p['g2'] for p in params]).reshape(nrel, 1, out)
  bt2s = jnp.stack([p['be2'] for p in params]).reshape(nrel, 1, out)

  seg = _make_seg_kernel(n, e, hid, nrel)
  degq = _make_deg_kernel(n, e, nrel)(dst_all * 16)
  degp = degq.reshape(nrel, NC, n, 16)
  dinv = _tc_comb(degp)
  h1, g1 = _tc_pre(features, w1s, dinv)
  p1 = seg(g1.reshape(nrel * n, hid), src_all, dst_all)
  agg1 = _tc_scale(p1, dinv)
  x1, gx1, losses = _tc_mid(agg1, h1, dinv, k21, b1s, gm1s, bt1s)
  p2 = seg(gx1.reshape(nrel * n, hid), src_all, dst_all)
  q = _tc_scale(p2, dinv)
  m, mu2, var2 = _tc_post(q, x1, k22, b2s, w2s)
  mb = _make_gather_kernel(nrel * n, 2 * out, nrel * b)(
      m.reshape(nrel * n, 2 * out), bidx)
  emb, lossm = _tc_final(mb.reshape(nrel, b, 2 * out), mu2, var2,
                         k22, b2s, gm2s, bt2s, losses)
  final = emb.transpose(1, 0, 2).reshape(b, nrel * out)
  return final, lossm[0, 0]


# speedup vs baseline: 13.8251x; 1.5063x over previous
import functools

import jax
import jax.numpy as jnp
from jax import lax
from jax.experimental import pallas as pl
from jax.experimental.pallas import tpu as pltpu
from jax.experimental.pallas import tpu_sc as plsc

EPS = 1e-5
NC = 2
NS = 16
NW = NC * NS


def _zero_fill(ref, nrows, ncols):
  def body(i, _):
    for c in range(ncols // 16):
      ref[i, pl.ds(c * 16, 16)] = jnp.zeros((16,), jnp.float32)
    return 0
  lax.fori_loop(0, nrows, body, 0)


def _row_partition(n):
  fl = (n // NS) & ~7
  extra = n - NS * fl
  zr = 8
  for z in range(128, 7, -8):
    if fl % z == 0:
      zr = z
      break
  assert extra % 8 == 0 and fl % zr == 0
  return fl, extra, zr


def _make_seg_kernel(n, e, d, nrel):
  ept = e // NW
  k = 80 if ept % 80 == 0 else ept
  assert ept % k == 0 and k <= 128 and k % 8 == 0, (e, ept, k)
  nchunk = ept // k
  npairs = max(0, (nchunk - 1) // 2)
  fl, extra, zr = _row_partition(n)
  mesh = plsc.VectorSubcoreMesh(core_axis_name="c", subcore_axis_name="s", num_cores=NC, num_subcores=NS)

  @functools.partial(
      pl.kernel,
      out_type=jax.ShapeDtypeStruct((nrel, NC, n, d), jnp.float32),
      mesh=mesh,
      scratch_types=[
          pltpu.VMEM((k,), jnp.int32),
          pltpu.VMEM((k,), jnp.int32),
          pltpu.VMEM((k,), jnp.int32),
          pltpu.VMEM((k,), jnp.int32),
          pltpu.VMEM((k, d), jnp.float32),
          pltpu.VMEM((k, d), jnp.float32),
          pltpu.VMEM((zr, d), jnp.float32),
          pltpu.VMEM_SHARED((n, d), jnp.float32),
          pltpu.SemaphoreType.DMA,
          pltpu.SemaphoreType.DMA,
      ],
  )
  def seg(g_hbm, src_hbm, dst_hbm, out_hbm, sidx0, sidx1, didx0, didx1,
          rows0, rows1, zv, acc, gsem0, gsem1):
    c = lax.axis_index("c")
    s = lax.axis_index("s")
    sidx = (sidx0, sidx1)
    didx = (didx0, didx1)
    rows = (rows0, rows1)
    gsem = (gsem0, gsem1)
    _zero_fill(zv, zr, d)
    for r in range(nrel):
      tile_base = r * e + (c * NS + s) * ept

      def load_idx(j, b):
        pltpu.sync_copy(src_hbm.at[pl.ds(tile_base + j * k, k)], sidx[b])
        pltpu.sync_copy(dst_hbm.at[pl.ds(tile_base + j * k, k)], didx[b])

      def start_gather(b):
        pltpu.async_copy(g_hbm.at[sidx[b]], rows[b], gsem[b])

      def wait_gather(b):
        pltpu.make_async_copy(g_hbm.at[sidx[b]], rows[b], gsem[b]).wait()

      def scatter(b):
        pltpu.sync_copy(rows[b], acc.at[didx[b]], add=True)

      for zc in range(fl // zr):
        pltpu.sync_copy(zv, acc.at[pl.ds(s * fl + zc * zr, zr)])
      if extra:
        @pl.when(s == NS - 1)
        def _():
          for zc in range(extra // 8):
            pltpu.sync_copy(zv.at[pl.ds(0, 8)],
                            acc.at[pl.ds(NS * fl + zc * 8, 8)])
      plsc.subcore_barrier()

      load_idx(0, 0)
      start_gather(0)

      def pair(jj, _):
        for b in (0, 1):
          j = jj * 2 + b
          load_idx(j + 1, 1 - b)
          start_gather(1 - b)
          wait_gather(b)
          scatter(b)
        return 0

      lax.fori_loop(0, npairs, pair, 0)
      for t in range(2 * npairs, nchunk):
        b = t % 2
        if t + 1 < nchunk:
          load_idx(t + 1, 1 - b)
          start_gather(1 - b)
        wait_gather(b)
        scatter(b)

      plsc.subcore_barrier()
      pltpu.sync_copy(acc.at[pl.ds(s * fl, fl)],
                      out_hbm.at[r, c, pl.ds(s * fl, fl)])
      if extra:
        @pl.when(s == NS - 1)
        def _():
          pltpu.sync_copy(acc.at[pl.ds(NS * fl, extra)],
                          out_hbm.at[r, c, pl.ds(NS * fl, extra)])
      plsc.subcore_barrier()

  return seg


def _make_deg_kernel(n, e, nrel):
  d = 16
  ept = e // NW
  k = 80 if ept % 80 == 0 else ept
  assert ept % k == 0 and k <= 128 and k % 8 == 0
  nchunk = ept // k
  fl, extra, zr = _row_partition(n)
  seg_sz = n * d
  mesh = plsc.VectorSubcoreMesh(core_axis_name="c", subcore_axis_name="s", num_cores=NC, num_subcores=NS)

  @functools.partial(
      pl.kernel,
      out_type=jax.ShapeDtypeStruct((nrel * NC * seg_sz,), jnp.float32),
      mesh=mesh,
      scratch_types=[
          pltpu.VMEM((k,), jnp.int32),
          pltpu.VMEM((k,), jnp.float32),
          pltpu.VMEM((zr * d,), jnp.float32),
          pltpu.VMEM_SHARED((seg_sz,), jnp.float32),
      ],
  )
  def deg(dst16_hbm, out_hbm, didx, ones_v, zv, acc):
    c = lax.axis_index("c")
    s = lax.axis_index("s")

    def fill(i, _):
      zv[pl.ds(i * 16, 16)] = jnp.zeros((16,), jnp.float32)
      return 0
    lax.fori_loop(0, zr * d // 16, fill, 0)

    def fill_ones(i, _):
      ones_v[pl.ds(i * 16, 16)] = jnp.ones((16,), jnp.float32)
      return 0
    lax.fori_loop(0, k // 16, fill_ones, 0)

    for r in range(nrel):
      for zc in range(fl // zr):
        pltpu.sync_copy(zv, acc.at[pl.ds((s * fl + zc * zr) * d, zr * d)])
      if extra:
        @pl.when(s == NS - 1)
        def _():
          for zc in range(extra * d // (8 * d)):
            pltpu.sync_copy(zv.at[pl.ds(0, 8 * d)],
                            acc.at[pl.ds((NS * fl + zc * 8) * d, 8 * d)])
      plsc.subcore_barrier()

      def chunk(j, _):
        base = r * e + (c * NS + s) * ept + j * k
        pltpu.sync_copy(dst16_hbm.at[pl.ds(base, k)], didx)
        pltpu.sync_copy(ones_v, acc.at[didx], add=True)
        return 0

      lax.fori_loop(0, nchunk, chunk, 0)
      plsc.subcore_barrier()
      pltpu.sync_copy(
          acc.at[pl.ds(s * fl * d, fl * d)],
          out_hbm.at[pl.ds((r * NC + c) * seg_sz + s * fl * d, fl * d)])
      if extra:
        @pl.when(s == NS - 1)
        def _():
          pltpu.sync_copy(
              acc.at[pl.ds(NS * fl * d, extra * d)],
              out_hbm.at[pl.ds((r * NC + c) * seg_sz + NS * fl * d,
                               extra * d)])
      plsc.subcore_barrier()

  return deg


def _make_gather_kernel(nrows_tab, d, nidx):
  per_tile = nidx // NW
  assert nidx % NW == 0 and per_tile % 8 == 0
  k = 128 if per_tile % 128 == 0 else per_tile
  assert per_tile % k == 0 and k <= 128
  nchunk = per_tile // k
  mesh = plsc.VectorSubcoreMesh(core_axis_name="c", subcore_axis_name="s", num_cores=NC, num_subcores=NS)

  @functools.partial(
      pl.kernel,
      out_type=jax.ShapeDtypeStruct((nidx, d), jnp.float32),
      mesh=mesh,
      scratch_types=[
          pltpu.VMEM((k,), jnp.int32),
          pltpu.VMEM((k, d), jnp.float32),
          pltpu.SemaphoreType.DMA,
      ],
  )
  def gat(tab_hbm, idx_hbm, out_hbm, idxv, rows, sem):
    c = lax.axis_index("c")
    s = lax.axis_index("s")
    wid = c * NS + s
    for q in range(nchunk):
      base = wid * per_tile + q * k
      pltpu.sync_copy(idx_hbm.at[pl.ds(base, k)], idxv)
      pltpu.async_copy(tab_hbm.at[idxv], rows, sem).wait()
      pltpu.sync_copy(rows, out_hbm.at[pl.ds(base, k)])

  return gat


def _tc_comb(degp):
  nrel, _, n, dw = degp.shape

  def body(degp_ref, dinv_ref):
    deg = degp_ref[0, 0, :, 0:1] + degp_ref[0, 1, :, 0:1]
    dinv_ref[0] = lax.rsqrt(jnp.maximum(deg, 1.0))

  return pl.pallas_call(
      body,
      grid=(nrel,),
      in_specs=[pl.BlockSpec((1, NC, n, dw), lambda r: (r, 0, 0, 0))],
      out_specs=[pl.BlockSpec((1, n, 1), lambda r: (r, 0, 0))],
      out_shape=[jax.ShapeDtypeStruct((nrel, n, 1), jnp.float32)],
  )(degp)[0]


def _tc_pre(x, w1s, dinv):
  n, feat = x.shape
  nrel, _, hid = w1s.shape

  def body(x_ref, w_ref, dinv_ref, h1_ref, g1_ref):
    h = jnp.dot(x_ref[...], w_ref[0], preferred_element_type=jnp.float32)
    h1_ref[0] = h
    g1_ref[0] = h * dinv_ref[0]

  return pl.pallas_call(
      body,
      grid=(nrel,),
      in_specs=[
          pl.BlockSpec((n, feat), lambda r: (0, 0)),
          pl.BlockSpec((1, feat, hid), lambda r: (r, 0, 0)),
          pl.BlockSpec((1, n, 1), lambda r: (r, 0, 0)),
      ],
      out_specs=[
          pl.BlockSpec((1, n, hid), lambda r: (r, 0, 0)),
          pl.BlockSpec((1, n, hid), lambda r: (r, 0, 0)),
      ],
      out_shape=[
          jax.ShapeDtypeStruct((nrel, n, hid), jnp.float32),
          jax.ShapeDtypeStruct((nrel, n, hid), jnp.float32),
      ],
  )(x, w1s, dinv)


def _tc_scale(p, dinv):
  nrel, _, n, d = p.shape

  def body(p_ref, dinv_ref, a_ref):
    p0 = p_ref[0]
    a_ref[0] = (p0[0] + p0[1]) * dinv_ref[0]

  return pl.pallas_call(
      body,
      grid=(nrel,),
      in_specs=[
          pl.BlockSpec((1, NC, n, d), lambda r: (r, 0, 0, 0)),
          pl.BlockSpec((1, n, 1), lambda r: (r, 0, 0)),
      ],
      out_specs=[pl.BlockSpec((1, n, d), lambda r: (r, 0, 0))],
      out_shape=[jax.ShapeDtypeStruct((nrel, n, d), jnp.float32)],
  )(p, dinv)[0]


def _tc_mid(agg1, h1, dinv, k21, b1s, gm1s, bt1s):
  nrel, n, hid = agg1.shape

  def body(agg_ref, h1_ref, dinv_ref, k2_ref, b_ref, gm_ref, bt_ref,
           x1_ref, gx1_ref, loss_ref):
    r = pl.program_id(0)
    agg = agg_ref[0]
    h = h1_ref[0]
    k2 = k2_ref[r, 0]
    out1 = agg + k2 * h + b_ref[0, 0][None, :]
    lap = (1.0 + k2) * h - agg
    loss_ref[r, 0] = jnp.mean(lap * lap)
    mu = jnp.mean(out1, axis=0, keepdims=True)
    var = jnp.mean((out1 - mu) ** 2, axis=0, keepdims=True)
    x1 = jnp.tanh(gm_ref[0, 0][None, :] * (out1 - mu)
                  / jnp.sqrt(var + EPS) + bt_ref[0, 0][None, :])
    x1_ref[0] = x1
    gx1_ref[0] = x1 * dinv_ref[0]

  return pl.pallas_call(
      body,
      grid=(nrel,),
      in_specs=[
          pl.BlockSpec((1, n, hid), lambda r: (r, 0, 0)),
          pl.BlockSpec((1, n, hid), lambda r: (r, 0, 0)),
          pl.BlockSpec((1, n, 1), lambda r: (r, 0, 0)),
          pl.BlockSpec(memory_space=pltpu.SMEM),
          pl.BlockSpec((1, 1, hid), lambda r: (r, 0, 0)),
          pl.BlockSpec((1, 1, hid), lambda r: (r, 0, 0)),
          pl.BlockSpec((1, 1, hid), lambda r: (r, 0, 0)),
      ],
      out_specs=[
          pl.BlockSpec((1, n, hid), lambda r: (r, 0, 0)),
          pl.BlockSpec((1, n, hid), lambda r: (r, 0, 0)),
          pl.BlockSpec(memory_space=pltpu.SMEM),
      ],
      out_shape=[
          jax.ShapeDtypeStruct((nrel, n, hid), jnp.float32),
          jax.ShapeDtypeStruct((nrel, n, hid), jnp.float32),
          jax.ShapeDtypeStruct((nrel, 1), jnp.float32),
      ],
  )(agg1, h1, dinv, k21, b1s, gm1s, bt1s)


def _tc_post(q, x1, k22, b2s, w2s):
  nrel, n, hid = q.shape
  out = w2s.shape[2]

  def body(q_ref, x1_ref, k2_ref, b_ref, w2_ref, m_ref, mu_ref, var_ref):
    r = pl.program_id(0)
    k2 = k2_ref[r, 0]
    w2 = w2_ref[0]
    agg2 = jnp.dot(q_ref[0], w2, preferred_element_type=jnp.float32)
    h2 = jnp.dot(x1_ref[0], w2, preferred_element_type=jnp.float32)
    out2 = agg2 + k2 * h2 + b_ref[0, 0][None, :]
    mu = jnp.mean(out2, axis=0, keepdims=True)
    var = jnp.mean((out2 - mu) ** 2, axis=0, keepdims=True)
    mu_ref[0] = mu
    var_ref[0] = var
    m_ref[0] = jnp.concatenate([agg2, h2], axis=1)

  return pl.pallas_call(
      body,
      grid=(nrel,),
      in_specs=[
          pl.BlockSpec((1, n, hid), lambda r: (r, 0, 0)),
          pl.BlockSpec((1, n, hid), lambda r: (r, 0, 0)),
          pl.BlockSpec(memory_space=pltpu.SMEM),
          pl.BlockSpec((1, 1, out), lambda r: (r, 0, 0)),
          pl.BlockSpec((1, hid, out), lambda r: (r, 0, 0)),
      ],
      out_specs=[
          pl.BlockSpec((1, n, 2 * out), lambda r: (r, 0, 0)),
          pl.BlockSpec((1, 1, out), lambda r: (r, 0, 0)),
          pl.BlockSpec((1, 1, out), lambda r: (r, 0, 0)),
      ],
      out_shape=[
          jax.ShapeDtypeStruct((nrel, n, 2 * out), jnp.float32),
          jax.ShapeDtypeStruct((nrel, 1, out), jnp.float32),
          jax.ShapeDtypeStruct((nrel, 1, out), jnp.float32),
      ],
  )(q, x1, k22, b2s, w2s)


def _tc_final(mb, mu2, var2, k22, b2s, gm2s, bt2s, losses):
  nrel, b, two_out = mb.shape
  out = two_out // 2

  def body(mb_ref, mu_ref, var_ref, k2_ref, b_ref, gm_ref, bt_ref, loss_ref,
           emb_ref, lm_ref):
    r = pl.program_id(0)
    m = mb_ref[0]
    agg2 = m[:, :out]
    h2 = m[:, out:]
    k2 = k2_ref[r, 0]
    out2 = agg2 + k2 * h2 + b_ref[0, 0][None, :]
    mu = mu_ref[0]
    var = var_ref[0]
    x2 = jnp.tanh(gm_ref[0, 0][None, :] * (out2 - mu)
                  / jnp.sqrt(var + EPS) + bt_ref[0, 0][None, :])
    mx = jnp.max(x2, axis=1, keepdims=True)
    lse = jnp.log(jnp.sum(jnp.exp(x2 - mx), axis=1, keepdims=True))
    emb_ref[0] = x2 - mx - lse

    @pl.when(r == 0)
    def _():
      lm_ref[0, 0] = 0.0
    lm_ref[0, 0] += loss_ref[r, 0] / nrel

  return pl.pallas_call(
      body,
      grid=(nrel,),
      in_specs=[
          pl.BlockSpec((1, b, two_out), lambda r: (r, 0, 0)),
          pl.BlockSpec((1, 1, out), lambda r: (r, 0, 0)),
          pl.BlockSpec((1, 1, out), lambda r: (r, 0, 0)),
          pl.BlockSpec(memory_space=pltpu.SMEM),
          pl.BlockSpec((1, 1, out), lambda r: (r, 0, 0)),
          pl.BlockSpec((1, 1, out), lambda r: (r, 0, 0)),
          pl.BlockSpec((1, 1, out), lambda r: (r, 0, 0)),
          pl.BlockSpec(memory_space=pltpu.SMEM),
      ],
      out_specs=[
          pl.BlockSpec((1, b, out), lambda r: (r, 0, 0)),
          pl.BlockSpec(memory_space=pltpu.SMEM),
      ],
      out_shape=[
          jax.ShapeDtypeStruct((nrel, b, out), jnp.float32),
          jax.ShapeDtypeStruct((1, 1), jnp.float32),
      ],
  )(mb, mu2, var2, k22, b2s, gm2s, bt2s, losses)


def kernel(features, edge_index0, edge_index1, edge_index2, batch_nodes,
           params):
  n, _ = features.shape
  e = edge_index0.shape[1]
  b = batch_nodes.shape[0]
  nrel = 3
  hid = params[0]['W1'].shape[1]
  out = params[0]['W2'].shape[1]

  i32 = jnp.int32
  src_all = jnp.concatenate([
      edge_index0[0].astype(i32),
      edge_index1[0].astype(i32) + n,
      edge_index2[0].astype(i32) + 2 * n,
  ])
  dst_all = jnp.concatenate([
      edge_index0[1].astype(i32),
      edge_index1[1].astype(i32),
      edge_index2[1].astype(i32),
  ])
  bidx = jnp.concatenate(
      [batch_nodes.astype(i32) + r * n for r in range(nrel)])

  w1s = jnp.stack([p['W1'] for p in params])
  b1s = jnp.stack([p['b1'] for p in params]).reshape(nrel, 1, hid)
  k21 = jnp.stack([p['k2_1'] for p in params]).reshape(nrel, 1)
  gm1s = jnp.stack([p['g1'] for p in params]).reshape(nrel, 1, hid)
  bt1s = jnp.stack([p['be1'] for p in params]).reshape(nrel, 1, hid)
  w2s = jnp.stack([p['W2'] for p in params])
  b2s = jnp.stack([p['b2'] for p in params]).reshape(nrel, 1, out)
  k22 = jnp.stack([p['k2_2'] for p in params]).reshape(nrel, 1)
  gm2s = jnp.stack([p['g2'] for p in params]).reshape(nrel, 1, out)
  bt2s = jnp.stack([p['be2'] for p in params]).reshape(nrel, 1, out)

  seg = _make_seg_kernel(n, e, hid, nrel)
  degq = _make_deg_kernel(n, e, nrel)(dst_all * 16)
  degp = degq.reshape(nrel, NC, n, 16)
  dinv = _tc_comb(degp)
  h1, g1 = _tc_pre(features, w1s, dinv)
  p1 = seg(g1.reshape(nrel * n, hid), src_all, dst_all)
  agg1 = _tc_scale(p1, dinv)
  x1, gx1, losses = _tc_mid(agg1, h1, dinv, k21, b1s, gm1s, bt1s)
  p2 = seg(gx1.reshape(nrel * n, hid), src_all, dst_all)
  q = _tc_scale(p2, dinv)
  m, mu2, var2 = _tc_post(q, x1, k22, b2s, w2s)
  mb = _make_gather_kernel(nrel * n, 2 * out, nrel * b)(
      m.reshape(nrel * n, 2 * out), bidx)
  emb, lossm = _tc_final(mb.reshape(nrel, b, 2 * out), mu2, var2,
                         k22, b2s, gm2s, bt2s, losses)
  final = emb.transpose(1, 0, 2).reshape(b, nrel * out)
  return final, lossm[0, 0]
